# baseline (speedup 1.0000x reference)
import functools

import jax
import jax.numpy as jnp
from jax import lax
from jax.experimental import pallas as pl
from jax.experimental.pallas import tpu as pltpu
from jax.experimental.pallas import tpu_sc as plsc

N_NODES = 10000
N_EDGES = 320000
D = 128

NC, NS, L = 2, 16, 16
NW = NC * NS
CHUNK = 128
CPT = 79
EPT = CPT * CHUNK
E_PAD = NW * EPT
NPAD = 10240
RPT = NPAD // NS
CPT0 = 95
CPT1 = 2 * CPT - CPT0
CPT_MAX = max(CPT0, CPT1)

_MESH = plsc.VectorSubcoreMesh(core_axis_name="c", subcore_axis_name="s",
                               num_cores=NC, num_subcores=NS)
_CP = pltpu.CompilerParams(needs_layout_passes=False)


@functools.partial(
    pl.kernel,
    out_type=jax.ShapeDtypeStruct((NC, NPAD), jnp.float32),
    mesh=_MESH,
    compiler_params=_CP,
    scratch_types=[
        pltpu.VMEM((CPT, 1, CHUNK), jnp.int32),
        pltpu.VMEM((CHUNK,), jnp.float32),
        pltpu.VMEM((RPT,), jnp.float32),
        pltpu.VMEM_SHARED((NPAD,), jnp.float32),
    ],
)
def _deg_kernel(dst_hbm, deg_hbm, idx_v, ones_v, zero_v, deg_sh):
    c = lax.axis_index("c")
    s = lax.axis_index("s")
    w = c * NS + s
    pltpu.sync_copy(dst_hbm.at[pl.ds(w * CPT, CPT)], idx_v)

    ones = jnp.ones((L,), jnp.float32)
    for i in range(CHUNK // L):
        ones_v[pl.ds(i * L, L)] = ones
    zeros = jnp.zeros((L,), jnp.float32)

    def _zero(i, _):
        zero_v[pl.ds(i * L, L)] = zeros
        return 0

    lax.fori_loop(0, RPT // L, _zero, 0)
    pltpu.sync_copy(zero_v, deg_sh.at[pl.ds(s * RPT, RPT)])
    plsc.subcore_barrier()

    def _accum(j, _):
        pltpu.sync_copy(ones_v, deg_sh.at[idx_v.at[j, 0]], add=True)
        return 0

    lax.fori_loop(0, CPT, _accum, 0)
    plsc.subcore_barrier()
    pltpu.sync_copy(deg_sh.at[pl.ds(s * RPT, RPT)],
                    deg_hbm.at[c, pl.ds(s * RPT, RPT)])


@functools.partial(
    pl.kernel,
    out_type=jax.ShapeDtypeStruct((NC, NPAD, D), jnp.float32),
    mesh=_MESH,
    compiler_params=_CP,
    scratch_types=[
        pltpu.VMEM((CPT_MAX, 1, CHUNK), jnp.int32),
        pltpu.VMEM((1, CHUNK), jnp.int32),
        pltpu.VMEM((1, CHUNK), jnp.int32),
        pltpu.VMEM((CHUNK, D), jnp.float32),
        pltpu.VMEM((CHUNK, D), jnp.float32),
        pltpu.VMEM_SHARED((NPAD, D), jnp.float32),
        pltpu.SemaphoreType.DMA,
        pltpu.SemaphoreType.DMA,
        pltpu.SemaphoreType.DMA,
        pltpu.SemaphoreType.DMA,
    ],
)
def _scatter_kernel(src_hbm, dst_hbm, hs0_hbm, hs1_hbm, acc_hbm,
                    src_v, dst_a, dst_b, rows_a, rows_b, acc_sh,
                    sem_a, sem_b, sem_da, sem_db):
    c = lax.axis_index("c")
    s = lax.axis_index("s")

    def _run(hs_hbm, base, cpt):
        pltpu.sync_copy(src_hbm.at[pl.ds(base, cpt)], src_v.at[pl.ds(0, cpt)])
        zeros = jnp.zeros((L,), jnp.float32)
        for r in range(CHUNK):
            for k in range(D // L):
                rows_a[r, pl.ds(k * L, L)] = zeros
        for k in range(RPT // CHUNK):
            pltpu.sync_copy(rows_a, acc_sh.at[pl.ds(s * RPT + k * CHUNK, CHUNK)])
        plsc.subcore_barrier()

        def _gather(j, buf, sem):
            pltpu.async_copy(hs_hbm.at[src_v.at[j, 0]], buf, sem)

        def _gwait(buf, sem):
            pltpu.make_async_copy(hs_hbm.at[src_v.at[0, 0]], buf, sem).wait()

        def _dload(j, buf, sem):
            pltpu.async_copy(dst_hbm.at[base + j], buf, sem)

        def _dwait(buf, sem):
            pltpu.make_async_copy(dst_hbm.at[0], buf, sem).wait()

        def _scatter(buf, dst_st):
            pltpu.sync_copy(buf, acc_sh.at[dst_st.at[0]], add=True)

        _gather(0, rows_a, sem_a)
        _dload(0, dst_a, sem_da)

        def _pair(i, _):
            j0 = 2 * i
            _dload(j0 + 1, dst_b, sem_db)
            _gwait(rows_a, sem_a)
            _gather(j0 + 1, rows_b, sem_b)
            _dwait(dst_a, sem_da)
            _scatter(rows_a, dst_a)
            _dload(j0 + 2, dst_a, sem_da)
            _gwait(rows_b, sem_b)
            _gather(j0 + 2, rows_a, sem_a)
            _dwait(dst_b, sem_db)
            _scatter(rows_b, dst_b)
            return 0

        lax.fori_loop(0, (cpt - 1) // 2, _pair, 0)
        _gwait(rows_a, sem_a)
        _dwait(dst_a, sem_da)
        _scatter(rows_a, dst_a)
        plsc.subcore_barrier()
        pltpu.sync_copy(acc_sh.at[pl.ds(s * RPT, RPT)],
                        acc_hbm.at[1 - c, pl.ds(s * RPT, RPT)])

    @pl.when(c == 0)
    def _():
        _run(hs0_hbm, s * CPT0, CPT0)

    @pl.when(c == 1)
    def _():
        _run(hs1_hbm, NS * CPT0 + s * CPT1, CPT1)


_ROWS = 1000


def _prep_body(x_ref, w_ref, dis_ref, hs0_ref, hs1_ref):
    h = jnp.dot(x_ref[...], w_ref[...], preferred_element_type=jnp.float32)
    hs = h * dis_ref[...]
    hs0_ref[...] = hs
    hs1_ref[...] = hs


def _final_body(a0_ref, a1_ref, hs_ref, dis_ref, b_ref, g_ref, be_ref, o_ref):
    t = (a0_ref[0] + a1_ref[0] + hs_ref[...]) * dis_ref[...] + b_ref[...]
    t = jnp.maximum(t, 0.0)
    mean = jnp.mean(t, axis=-1, keepdims=True)
    var = jnp.mean(jnp.square(t - mean), axis=-1, keepdims=True)
    o_ref[...] = (t - mean) * lax.rsqrt(var + 1e-5) * g_ref[...] + be_ref[...]


def kernel(x, edge_index, W, b, gamma, beta):
    src = edge_index[0].astype(jnp.int32)
    dst = edge_index[1].astype(jnp.int32)
    pad = E_PAD - N_EDGES
    src_p = jnp.concatenate([src, jnp.zeros((pad,), jnp.int32)])
    dst_p = jnp.concatenate([dst, jnp.full((pad,), N_NODES, jnp.int32)])
    src3d = src_p.reshape(NW * CPT, 1, CHUNK)
    dst3d = dst_p.reshape(NW * CPT, 1, CHUNK)

    deg2 = _deg_kernel(dst3d)
    deg = deg2[0, :N_NODES] + deg2[1, :N_NODES] + 1.0
    dis = lax.rsqrt(deg).reshape(N_NODES, 1)

    grid = N_NODES // _ROWS
    hs, hs_copy = pl.pallas_call(
        _prep_body,
        grid=(grid,),
        in_specs=[
            pl.BlockSpec((_ROWS, D), lambda i: (i, 0)),
            pl.BlockSpec((D, D), lambda i: (0, 0)),
            pl.BlockSpec((_ROWS, 1), lambda i: (i, 0)),
        ],
        out_specs=[pl.BlockSpec((_ROWS, D), lambda i: (i, 0)),
                   pl.BlockSpec((_ROWS, D), lambda i: (i, 0))],
        out_shape=[jax.ShapeDtypeStruct((N_NODES, D), jnp.float32),
                   jax.ShapeDtypeStruct((N_NODES, D), jnp.float32)],
    )(x, W, dis)

    acc = _scatter_kernel(src3d, dst3d, hs, hs_copy)

    out = pl.pallas_call(
        _final_body,
        grid=(grid,),
        in_specs=[
            pl.BlockSpec((1, _ROWS, D), lambda i: (0, i, 0)),
            pl.BlockSpec((1, _ROWS, D), lambda i: (1, i, 0)),
            pl.BlockSpec((_ROWS, D), lambda i: (i, 0)),
            pl.BlockSpec((_ROWS, 1), lambda i: (i, 0)),
            pl.BlockSpec((1, D), lambda i: (0, 0)),
            pl.BlockSpec((1, D), lambda i: (0, 0)),
            pl.BlockSpec((1, D), lambda i: (0, 0)),
        ],
        out_specs=pl.BlockSpec((_ROWS, D), lambda i: (i, 0)),
        out_shape=jax.ShapeDtypeStruct((N_NODES, D), jnp.float32),
    )(acc, acc, hs, dis, b.reshape(1, D), gamma.reshape(1, D),
      beta.reshape(1, D))
    return out


# speedup vs baseline: 1.0601x; 1.0601x over previous
import functools

import jax
import jax.numpy as jnp
from jax import lax
from jax.experimental import pallas as pl
from jax.experimental.pallas import tpu as pltpu
from jax.experimental.pallas import tpu_sc as plsc

N_NODES = 10000
N_EDGES = 320000
D = 128

NC, NS, L = 2, 16, 16
NW = NC * NS
CHUNK = 128
CPT = 79
EPT = CPT * CHUNK
E_PAD = NW * EPT
NPAD = 10240
RPT = NPAD // NS
CPT0 = 109
CPT1 = 2 * CPT - CPT0
CPT_MAX = max(CPT0, CPT1)

_MESH = plsc.VectorSubcoreMesh(core_axis_name="c", subcore_axis_name="s",
                               num_cores=NC, num_subcores=NS)
_CP = pltpu.CompilerParams(needs_layout_passes=False)


@functools.partial(
    pl.kernel,
    out_type=jax.ShapeDtypeStruct((NC, NPAD), jnp.float32),
    mesh=_MESH,
    compiler_params=_CP,
    scratch_types=[
        pltpu.VMEM((CPT, 1, CHUNK), jnp.int32),
        pltpu.VMEM((CHUNK,), jnp.float32),
        pltpu.VMEM((RPT,), jnp.float32),
        pltpu.VMEM_SHARED((NPAD,), jnp.float32),
    ],
)
def _deg_kernel(dst_hbm, deg_hbm, idx_v, ones_v, zero_v, deg_sh):
    c = lax.axis_index("c")
    s = lax.axis_index("s")
    w = c * NS + s
    pltpu.sync_copy(dst_hbm.at[pl.ds(w * CPT, CPT)], idx_v)

    ones = jnp.ones((L,), jnp.float32)
    for i in range(CHUNK // L):
        ones_v[pl.ds(i * L, L)] = ones
    zeros = jnp.zeros((L,), jnp.float32)

    def _zero(i, _):
        zero_v[pl.ds(i * L, L)] = zeros
        return 0

    lax.fori_loop(0, RPT // L, _zero, 0)
    pltpu.sync_copy(zero_v, deg_sh.at[pl.ds(s * RPT, RPT)])
    plsc.subcore_barrier()

    def _accum(j, _):
        pltpu.sync_copy(ones_v, deg_sh.at[idx_v.at[j, 0]], add=True)
        return 0

    lax.fori_loop(0, CPT, _accum, 0)
    plsc.subcore_barrier()
    pltpu.sync_copy(deg_sh.at[pl.ds(s * RPT, RPT)],
                    deg_hbm.at[c, pl.ds(s * RPT, RPT)])


@functools.partial(
    pl.kernel,
    out_type=jax.ShapeDtypeStruct((NC, NPAD, D), jnp.float32),
    mesh=_MESH,
    compiler_params=_CP,
    scratch_types=[
        pltpu.VMEM((CPT_MAX, 1, CHUNK), jnp.int32),
        pltpu.VMEM((1, CHUNK), jnp.int32),
        pltpu.VMEM((1, CHUNK), jnp.int32),
        pltpu.VMEM((CHUNK, D), jnp.float32),
        pltpu.VMEM((CHUNK, D), jnp.float32),
        pltpu.VMEM_SHARED((NPAD, D), jnp.float32),
        pltpu.SemaphoreType.DMA,
        pltpu.SemaphoreType.DMA,
        pltpu.SemaphoreType.DMA,
        pltpu.SemaphoreType.DMA,
    ],
)
def _scatter_kernel(src_hbm, dst_hbm, hs0_hbm, hs1_hbm, acc_hbm,
                    src_v, dst_a, dst_b, rows_a, rows_b, acc_sh,
                    sem_a, sem_b, sem_da, sem_db):
    c = lax.axis_index("c")
    s = lax.axis_index("s")

    def _run(hs_hbm, base, cpt):
        pltpu.sync_copy(src_hbm.at[pl.ds(base, cpt)], src_v.at[pl.ds(0, cpt)])
        zeros = jnp.zeros((L,), jnp.float32)
        for r in range(CHUNK):
            for k in range(D // L):
                rows_a[r, pl.ds(k * L, L)] = zeros
        for k in range(RPT // CHUNK):
            pltpu.sync_copy(rows_a, acc_sh.at[pl.ds(s * RPT + k * CHUNK, CHUNK)])
        plsc.subcore_barrier()

        def _gather(j, buf, sem):
            pltpu.async_copy(hs_hbm.at[src_v.at[j, 0]], buf, sem)

        def _gwait(buf, sem):
            pltpu.make_async_copy(hs_hbm.at[src_v.at[0, 0]], buf, sem).wait()

        def _dload(j, buf, sem):
            pltpu.async_copy(dst_hbm.at[base + j], buf, sem)

        def _dwait(buf, sem):
            pltpu.make_async_copy(dst_hbm.at[0], buf, sem).wait()

        def _scatter(buf, dst_st):
            pltpu.sync_copy(buf, acc_sh.at[dst_st.at[0]], add=True)

        _gather(0, rows_a, sem_a)
        _dload(0, dst_a, sem_da)

        def _pair(i, _):
            j0 = 2 * i
            _dload(j0 + 1, dst_b, sem_db)
            _gwait(rows_a, sem_a)
            _gather(j0 + 1, rows_b, sem_b)
            _dwait(dst_a, sem_da)
            _scatter(rows_a, dst_a)
            _dload(j0 + 2, dst_a, sem_da)
            _gwait(rows_b, sem_b)
            _gather(j0 + 2, rows_a, sem_a)
            _dwait(dst_b, sem_db)
            _scatter(rows_b, dst_b)
            return 0

        lax.fori_loop(0, (cpt - 1) // 2, _pair, 0)
        _gwait(rows_a, sem_a)
        _dwait(dst_a, sem_da)
        _scatter(rows_a, dst_a)
        plsc.subcore_barrier()
        pltpu.sync_copy(acc_sh.at[pl.ds(s * RPT, RPT)],
                        acc_hbm.at[1 - c, pl.ds(s * RPT, RPT)])

    @pl.when(c == 0)
    def _():
        _run(hs0_hbm, s * CPT0, CPT0)

    @pl.when(c == 1)
    def _():
        _run(hs1_hbm, NS * CPT0 + s * CPT1, CPT1)


_ROWS = 1000


def _prep_body(x_ref, w_ref, dis_ref, hs0_ref, hs1_ref):
    h = jnp.dot(x_ref[...], w_ref[...], preferred_element_type=jnp.float32)
    hs = h * dis_ref[...]
    hs0_ref[...] = hs
    hs1_ref[...] = hs


def _final_body(a0_ref, a1_ref, hs_ref, dis_ref, b_ref, g_ref, be_ref, o_ref):
    t = (a0_ref[0] + a1_ref[0] + hs_ref[...]) * dis_ref[...] + b_ref[...]
    t = jnp.maximum(t, 0.0)
    mean = jnp.mean(t, axis=-1, keepdims=True)
    var = jnp.mean(jnp.square(t - mean), axis=-1, keepdims=True)
    o_ref[...] = (t - mean) * lax.rsqrt(var + 1e-5) * g_ref[...] + be_ref[...]


def kernel(x, edge_index, W, b, gamma, beta):
    src = edge_index[0].astype(jnp.int32)
    dst = edge_index[1].astype(jnp.int32)
    pad = E_PAD - N_EDGES
    src_p = jnp.concatenate([src, jnp.zeros((pad,), jnp.int32)])
    dst_p = jnp.concatenate([dst, jnp.full((pad,), N_NODES, jnp.int32)])
    src3d = src_p.reshape(NW * CPT, 1, CHUNK)
    dst3d = dst_p.reshape(NW * CPT, 1, CHUNK)

    deg2 = _deg_kernel(dst3d)
    deg = deg2[0, :N_NODES] + deg2[1, :N_NODES] + 1.0
    dis = lax.rsqrt(deg).reshape(N_NODES, 1)

    grid = N_NODES // _ROWS
    hs, hs_copy = pl.pallas_call(
        _prep_body,
        grid=(grid,),
        in_specs=[
            pl.BlockSpec((_ROWS, D), lambda i: (i, 0)),
            pl.BlockSpec((D, D), lambda i: (0, 0)),
            pl.BlockSpec((_ROWS, 1), lambda i: (i, 0)),
        ],
        out_specs=[pl.BlockSpec((_ROWS, D), lambda i: (i, 0)),
                   pl.BlockSpec((_ROWS, D), lambda i: (i, 0))],
        out_shape=[jax.ShapeDtypeStruct((N_NODES, D), jnp.float32),
                   jax.ShapeDtypeStruct((N_NODES, D), jnp.float32)],
    )(x, W, dis)

    acc = _scatter_kernel(src3d, dst3d, hs, hs_copy)

    out = pl.pallas_call(
        _final_body,
        grid=(grid,),
        in_specs=[
            pl.BlockSpec((1, _ROWS, D), lambda i: (0, i, 0)),
            pl.BlockSpec((1, _ROWS, D), lambda i: (1, i, 0)),
            pl.BlockSpec((_ROWS, D), lambda i: (i, 0)),
            pl.BlockSpec((_ROWS, 1), lambda i: (i, 0)),
            pl.BlockSpec((1, D), lambda i: (0, 0)),
            pl.BlockSpec((1, D), lambda i: (0, 0)),
            pl.BlockSpec((1, D), lambda i: (0, 0)),
        ],
        out_specs=pl.BlockSpec((_ROWS, D), lambda i: (i, 0)),
        out_shape=jax.ShapeDtypeStruct((N_NODES, D), jnp.float32),
    )(acc, acc, hs, dis, b.reshape(1, D), gamma.reshape(1, D),
      beta.reshape(1, D))
    return out
